# baseline (speedup 1.0000x reference)
import functools

import jax
import jax.numpy as jnp
from jax import lax
from jax.experimental import pallas as pl
from jax.experimental.pallas import tpu as pltpu
from jax.experimental.pallas import tpu_sc as plsc

N_NODES = 10000
N_EDGES = 320000
D_IN = 1422
D_EDGE = 8
D_H = 128
N_BLOCK = 5

NC = 2
NS = 16
NW = NC * NS
EPW = N_EDGES // NW
CHUNK = 40
NCHUNK = EPW // CHUNK
N_PAD = 10240
ROWS_PER_TILE = N_PAD // NS
LANE = 16
NSL = D_H // LANE


def _sc_agg_body(y1, y2, e_i, eidx, out,
                 idx0, idxd0, r10, r20, re0,
                 idx1, idxd1, r11, r21, re1,
                 idx2, idxd2, r12, r22, re2,
                 aggsh,
                 semL0, semG0, semL1, semG1, semL2, semG2):
    c = lax.axis_index("c")
    s = lax.axis_index("s")
    w = s * NC + c

    sets = (
        (idx0, idxd0, r10, r20, re0, semL0, semG0),
        (idx1, idxd1, r11, r21, re1, semL1, semG1),
        (idx2, idxd2, r12, r22, re2, semL2, semG2),
    )

    def zfill(i, carry):
        for j in range(NSL):
            re0[i, pl.ds(j * LANE, LANE)] = jnp.zeros((LANE,), jnp.float32)
        return carry

    lax.fori_loop(0, CHUNK, zfill, 0)
    for k in range(ROWS_PER_TILE // CHUNK):
        pltpu.sync_copy(re0, aggsh.at[pl.ds(s * ROWS_PER_TILE + k * CHUNK, CHUNK)])
    plsc.subcore_barrier()

    def stage1(k, st):
        idx_v, idxd_v, _, _, re_v, semL, _ = st
        base = w * EPW + k * CHUNK
        pltpu.async_copy(eidx.at[w, k, 0], idx_v, semL)
        pltpu.async_copy(eidx.at[w, k, 1], idxd_v, semL)
        pltpu.async_copy(e_i.at[pl.ds(base, CHUNK)], re_v, semL)

    def wait1(k, st):
        idx_v, idxd_v, _, _, re_v, semL, _ = st
        base = w * EPW + k * CHUNK
        pltpu.make_async_copy(eidx.at[w, k, 0], idx_v, semL).wait()
        pltpu.make_async_copy(eidx.at[w, k, 1], idxd_v, semL).wait()
        pltpu.make_async_copy(e_i.at[pl.ds(base, CHUNK)], re_v, semL).wait()

    def gathers(st):
        idx_v, idxd_v, r1_v, r2_v, _, _, semG = st
        pltpu.async_copy(y1.at[idx_v], r1_v, semG)
        pltpu.async_copy(y2.at[idxd_v], r2_v, semG)

    def wait_g(st):
        idx_v, idxd_v, r1_v, r2_v, _, _, semG = st
        pltpu.make_async_copy(y1.at[idx_v], r1_v, semG).wait()
        pltpu.make_async_copy(y2.at[idxd_v], r2_v, semG).wait()

    def compute(st):
        idx_v, idxd_v, r1_v, r2_v, re_v, _, _ = st

        def crow(i, cc):
            for j in range(NSL):
                sl = pl.ds(j * LANE, LANE)
                r1_v[i, sl] = jnp.maximum(
                    r1_v[i, sl] + r2_v[i, sl] + re_v[i, sl], 0.0
                )
            return cc

        lax.fori_loop(0, CHUNK, crow, 0, unroll=4)

    def scatter(st):
        idx_v, idxd_v, r1_v, r2_v, re_v, _, _ = st
        pltpu.sync_copy(r1_v, aggsh.at[idxd_v], add=True)

    stage1(0, sets[0])
    stage1(1, sets[1])
    stage1(2, sets[2])
    wait1(0, sets[0])
    gathers(sets[0])
    wait_g(sets[0])

    def body(t, carry):
        k0 = 3 * t

        def g_step(k, st):
            @pl.when(k < NCHUNK)
            def _():
                wait1(k, st)
                gathers(st)

        def gw_step(k, st):
            @pl.when(k < NCHUNK)
            def _():
                wait_g(st)

        def c_step(k, st):
            @pl.when(k < NCHUNK)
            def _():
                compute(st)

        def s_step(k, st):
            @pl.when(k < NCHUNK)
            def _():
                scatter(st)

        def l_step(k, st):
            @pl.when(k < NCHUNK)
            def _():
                stage1(k, st)

        def handle(k, st_x, st_y):
            g_step(k + 1, st_y)
            c_step(k, st_x)
            gw_step(k + 1, st_y)
            s_step(k, st_x)
            l_step(k + 3, st_x)

        handle(k0, sets[0], sets[1])
        handle(k0 + 1, sets[1], sets[2])
        handle(k0 + 2, sets[2], sets[0])
        return carry

    lax.fori_loop(0, (NCHUNK + 2) // 3, body, 0)
    plsc.subcore_barrier()
    pltpu.sync_copy(
        aggsh.at[pl.ds(s * ROWS_PER_TILE, ROWS_PER_TILE)],
        out.at[pl.ds(c * N_PAD + s * ROWS_PER_TILE, ROWS_PER_TILE)],
    )


_sc_agg = functools.partial(
    pl.kernel,
    out_type=jax.ShapeDtypeStruct((NC * N_PAD, D_H), jnp.float32),
    mesh=plsc.VectorSubcoreMesh(
        core_axis_name="c", subcore_axis_name="s", num_cores=NC, num_subcores=NS
    ),
    scratch_types=(
        [
            pltpu.VMEM((CHUNK,), jnp.int32),
            pltpu.VMEM((CHUNK,), jnp.int32),
            pltpu.VMEM((CHUNK, D_H), jnp.float32),
            pltpu.VMEM((CHUNK, D_H), jnp.float32),
            pltpu.VMEM((CHUNK, D_H), jnp.float32),
        ] * 3
        + [pltpu.VMEM_SHARED((N_PAD, D_H), jnp.float32)]
        + [pltpu.SemaphoreType.DMA] * 6
    ),
)(_sc_agg_body)


def _proj_body(a_ref, w_ref, b_ref, o_ref):
    acc = jnp.dot(a_ref[...], w_ref[...], preferred_element_type=jnp.float32)
    o_ref[...] = jnp.maximum(acc + b_ref[...], 0.0)


def _proj(atom, w_in, b_in):
    bm = 1000
    grid = (N_NODES // bm,)
    return pl.pallas_call(
        _proj_body,
        grid=grid,
        in_specs=[
            pl.BlockSpec((bm, D_IN), lambda i: (i, 0)),
            pl.BlockSpec((D_IN, D_H), lambda i: (0, 0)),
            pl.BlockSpec((1, D_H), lambda i: (0, 0)),
        ],
        out_specs=pl.BlockSpec((bm, D_H), lambda i: (i, 0)),
        out_shape=jax.ShapeDtypeStruct((N_NODES, D_H), jnp.float32),
    )(atom, w_in, b_in.reshape(1, D_H))


def _edge_proj_body(bond_ref, w_ref, o_ref):
    o_ref[...] = jnp.dot(bond_ref[...], w_ref[...], preferred_element_type=jnp.float32)


def _edge_proj(bond2, w2):
    bm = 2000
    nrow = N_EDGES // 16
    grid = (nrow // bm,)
    return pl.pallas_call(
        _edge_proj_body,
        grid=grid,
        in_specs=[
            pl.BlockSpec((bm, D_H), lambda j: (j, 0)),
            pl.BlockSpec((D_H, 16 * D_H), lambda j: (0, 0)),
        ],
        out_specs=pl.BlockSpec((bm, 16 * D_H), lambda j: (j, 0)),
        out_shape=jax.ShapeDtypeStruct((nrow, 16 * D_H), jnp.float32),
    )(bond2, w2)


def _y12_body(x_ref, a_ref, b_ref, bm_ref, y1_ref, y2_ref):
    x = x_ref[...]
    y1_ref[...] = jnp.dot(x, a_ref[...], preferred_element_type=jnp.float32)
    y2_ref[...] = (
        jnp.dot(x, b_ref[...], preferred_element_type=jnp.float32) + bm_ref[...]
    )


def _y12(x, a, b, bm_i):
    bmr = 2000
    grid = (N_NODES // bmr,)
    return pl.pallas_call(
        _y12_body,
        grid=grid,
        in_specs=[
            pl.BlockSpec((bmr, D_H), lambda i: (i, 0)),
            pl.BlockSpec((D_H, D_H), lambda i: (0, 0)),
            pl.BlockSpec((D_H, D_H), lambda i: (0, 0)),
            pl.BlockSpec((1, D_H), lambda i: (0, 0)),
        ],
        out_specs=[
            pl.BlockSpec((bmr, D_H), lambda i: (i, 0)),
            pl.BlockSpec((bmr, D_H), lambda i: (i, 0)),
        ],
        out_shape=[
            jax.ShapeDtypeStruct((N_NODES, D_H), jnp.float32),
            jax.ShapeDtypeStruct((N_NODES, D_H), jnp.float32),
        ],
    )(x, a, b, bm_i.reshape(1, D_H))


def _update_body(x_ref, p0_ref, p1_ref, wu_ref, bu_ref, o_ref):
    agg = p0_ref[...] + p1_ref[...]
    acc = jnp.dot(agg, wu_ref[...], preferred_element_type=jnp.float32)
    o_ref[...] = jnp.maximum(x_ref[...] + acc + bu_ref[...], 0.0)


def _update(x, p0, p1, wu, bu):
    bmr = 2000
    grid = (N_NODES // bmr,)
    return pl.pallas_call(
        _update_body,
        grid=grid,
        in_specs=[
            pl.BlockSpec((bmr, D_H), lambda i: (i, 0)),
            pl.BlockSpec((bmr, D_H), lambda i: (i, 0)),
            pl.BlockSpec((bmr, D_H), lambda i: (i, 0)),
            pl.BlockSpec((D_H, D_H), lambda i: (0, 0)),
            pl.BlockSpec((1, D_H), lambda i: (0, 0)),
        ],
        out_specs=pl.BlockSpec((bmr, D_H), lambda i: (i, 0)),
        out_shape=jax.ShapeDtypeStruct((N_NODES, D_H), jnp.float32),
    )(x, p0, p1, wu, bu.reshape(1, D_H))


def _score_body(x_ref, w_ref, b_ref, o_ref):
    acc = jnp.dot(x_ref[...], w_ref[...], preferred_element_type=jnp.float32)
    o_ref[...] = jax.nn.sigmoid(acc + b_ref[...])


def _score(x, w_pad, b_pad):
    return pl.pallas_call(
        _score_body,
        in_specs=[
            pl.BlockSpec((N_NODES, D_H), lambda: (0, 0)),
            pl.BlockSpec((D_H, 8), lambda: (0, 0)),
            pl.BlockSpec((1, 8), lambda: (0, 0)),
        ],
        out_specs=pl.BlockSpec((N_NODES, 8), lambda: (0, 0)),
        out_shape=jax.ShapeDtypeStruct((N_NODES, 8), jnp.float32),
    )(x, w_pad, b_pad)


def kernel(atom_feature, edge_index, bond_feature, node2graph, b_factor,
           W_in, b_in, Wm, bm, Wu, bu, W_out, b_out):
    eidx = jnp.stack(
        [
            edge_index[0].reshape(NW, NCHUNK, CHUNK),
            edge_index[1].reshape(NW, NCHUNK, CHUNK),
        ],
        axis=2,
    )

    x = _proj(atom_feature, W_in, b_in)

    bond2 = bond_feature.reshape(N_EDGES // 16, 16 * D_EDGE)
    eye16 = jnp.eye(16, dtype=jnp.float32)
    e_all = [
        _edge_proj(bond2, jnp.kron(eye16, Wm[i, 2 * D_H:, :])).reshape(
            N_EDGES, D_H
        )
        for i in range(N_BLOCK)
    ]

    for i in range(N_BLOCK):
        y1, y2 = _y12(x, Wm[i, :D_H], Wm[i, D_H:2 * D_H], bm[i])
        parts = _sc_agg(y1, y2, e_all[i], eidx)
        x = _update(x, parts[:N_NODES], parts[N_PAD:N_PAD + N_NODES], Wu[i], bu[i])

    w_pad = jnp.pad(W_out, ((0, 0), (0, 7)))
    b_pad = jnp.pad(b_out.reshape(1, 1), ((0, 0), (0, 7)))
    out8 = _score(x, w_pad, b_pad)
    output = out8[:, 0]
    labels = b_factor
    return (output, labels)


# speedup vs baseline: 1.2864x; 1.2864x over previous
import functools

import jax
import jax.numpy as jnp
from jax import lax
from jax.experimental import pallas as pl
from jax.experimental.pallas import tpu as pltpu
from jax.experimental.pallas import tpu_sc as plsc

N_NODES = 10000
N_EDGES = 320000
D_IN = 1422
D_EDGE = 8
D_H = 128
N_BLOCK = 5

NC = 2
NS = 16
NW = NC * NS
EPW = N_EDGES // NW
CHUNK = 40
NCHUNK = EPW // CHUNK
N_PAD = 10240
ROWS_PER_TILE = N_PAD // NS
LANE = 16
NSL = D_H // LANE


def _sc_agg_body(y1, y2, e_i, eidx, out,
                 idx0, idxd0, r10, r20, re0,
                 idx1, idxd1, r11, r21, re1,
                 idx2, idxd2, r12, r22, re2,
                 aggsh,
                 semL0, semG0, semL1, semG1, semL2, semG2):
    c = lax.axis_index("c")
    s = lax.axis_index("s")
    w = s * NC + c

    sets = (
        (idx0, idxd0, r10, r20, re0, semL0, semG0),
        (idx1, idxd1, r11, r21, re1, semL1, semG1),
        (idx2, idxd2, r12, r22, re2, semL2, semG2),
    )

    def zfill(i, carry):
        for j in range(NSL):
            re0[i, pl.ds(j * LANE, LANE)] = jnp.zeros((LANE,), jnp.float32)
        return carry

    lax.fori_loop(0, CHUNK, zfill, 0)
    for k in range(ROWS_PER_TILE // CHUNK):
        pltpu.sync_copy(re0, aggsh.at[pl.ds(s * ROWS_PER_TILE + k * CHUNK, CHUNK)])
    plsc.subcore_barrier()

    def stage1(k, st):
        idx_v, idxd_v, _, _, re_v, semL, _ = st
        base = w * EPW + k * CHUNK
        pltpu.async_copy(eidx.at[w, k, 0], idx_v, semL)
        pltpu.async_copy(eidx.at[w, k, 1], idxd_v, semL)
        pltpu.async_copy(e_i.at[pl.ds(base, CHUNK)], re_v, semL)

    def wait1(k, st):
        idx_v, idxd_v, _, _, re_v, semL, _ = st
        base = w * EPW + k * CHUNK
        pltpu.make_async_copy(eidx.at[w, k, 0], idx_v, semL).wait()
        pltpu.make_async_copy(eidx.at[w, k, 1], idxd_v, semL).wait()
        pltpu.make_async_copy(e_i.at[pl.ds(base, CHUNK)], re_v, semL).wait()

    def gathers(st):
        idx_v, idxd_v, r1_v, r2_v, _, _, semG = st
        pltpu.async_copy(y1.at[idx_v], r1_v, semG)
        pltpu.async_copy(y2.at[idxd_v], r2_v, semG)

    def wait_g(st):
        idx_v, idxd_v, r1_v, r2_v, _, _, semG = st
        pltpu.make_async_copy(y1.at[idx_v], r1_v, semG).wait()
        pltpu.make_async_copy(y2.at[idxd_v], r2_v, semG).wait()

    def compute(st):
        idx_v, idxd_v, r1_v, r2_v, re_v, _, _ = st

        def crow(i, cc):
            for j in range(NSL):
                sl = pl.ds(j * LANE, LANE)
                r1_v[i, sl] = jnp.maximum(
                    r1_v[i, sl] + r2_v[i, sl] + re_v[i, sl], 0.0
                )
            return cc

        lax.fori_loop(0, CHUNK, crow, 0)

    def scatter(st):
        idx_v, idxd_v, r1_v, r2_v, re_v, _, _ = st
        pltpu.sync_copy(r1_v, aggsh.at[idxd_v], add=True)

    stage1(0, sets[0])
    stage1(1, sets[1])
    stage1(2, sets[2])
    wait1(0, sets[0])
    gathers(sets[0])
    wait_g(sets[0])

    def body(t, carry):
        k0 = 3 * t

        def g_step(k, st):
            @pl.when(k < NCHUNK)
            def _():
                wait1(k, st)
                gathers(st)

        def gw_step(k, st):
            @pl.when(k < NCHUNK)
            def _():
                wait_g(st)

        def c_step(k, st):
            @pl.when(k < NCHUNK)
            def _():
                compute(st)

        def s_step(k, st):
            @pl.when(k < NCHUNK)
            def _():
                scatter(st)

        def l_step(k, st):
            @pl.when(k < NCHUNK)
            def _():
                stage1(k, st)

        def handle(k, st_x, st_y):
            g_step(k + 1, st_y)
            c_step(k, st_x)
            gw_step(k + 1, st_y)
            s_step(k, st_x)
            l_step(k + 3, st_x)

        handle(k0, sets[0], sets[1])
        handle(k0 + 1, sets[1], sets[2])
        handle(k0 + 2, sets[2], sets[0])
        return carry

    lax.fori_loop(0, (NCHUNK + 2) // 3, body, 0)
    plsc.subcore_barrier()
    pltpu.sync_copy(
        aggsh.at[pl.ds(s * ROWS_PER_TILE, ROWS_PER_TILE)],
        out.at[pl.ds(c * N_PAD + s * ROWS_PER_TILE, ROWS_PER_TILE)],
    )


_sc_agg = functools.partial(
    pl.kernel,
    out_type=jax.ShapeDtypeStruct((NC * N_PAD, D_H), jnp.float32),
    mesh=plsc.VectorSubcoreMesh(
        core_axis_name="c", subcore_axis_name="s", num_cores=NC, num_subcores=NS
    ),
    scratch_types=(
        [
            pltpu.VMEM((CHUNK,), jnp.int32),
            pltpu.VMEM((CHUNK,), jnp.int32),
            pltpu.VMEM((CHUNK, D_H), jnp.float32),
            pltpu.VMEM((CHUNK, D_H), jnp.float32),
            pltpu.VMEM((CHUNK, D_H), jnp.float32),
        ] * 3
        + [pltpu.VMEM_SHARED((N_PAD, D_H), jnp.float32)]
        + [pltpu.SemaphoreType.DMA] * 6
    ),
)(_sc_agg_body)


def _proj_body(a_ref, w_ref, b_ref, o_ref):
    acc = jnp.dot(a_ref[...], w_ref[...], preferred_element_type=jnp.float32)
    o_ref[...] = jnp.maximum(acc + b_ref[...], 0.0)


def _proj(atom, w_in, b_in):
    bm = 1000
    grid = (N_NODES // bm,)
    return pl.pallas_call(
        _proj_body,
        grid=grid,
        in_specs=[
            pl.BlockSpec((bm, D_IN), lambda i: (i, 0)),
            pl.BlockSpec((D_IN, D_H), lambda i: (0, 0)),
            pl.BlockSpec((1, D_H), lambda i: (0, 0)),
        ],
        out_specs=pl.BlockSpec((bm, D_H), lambda i: (i, 0)),
        out_shape=jax.ShapeDtypeStruct((N_NODES, D_H), jnp.float32),
    )(atom, w_in, b_in.reshape(1, D_H))


def _edge_proj_body(bond_ref, w_ref, o_ref):
    o_ref[...] = jnp.dot(bond_ref[...], w_ref[...], preferred_element_type=jnp.float32)


def _edge_proj(bond2, w2):
    bm = 2000
    nrow = N_EDGES // 16
    grid = (nrow // bm,)
    return pl.pallas_call(
        _edge_proj_body,
        grid=grid,
        in_specs=[
            pl.BlockSpec((bm, D_H), lambda j: (j, 0)),
            pl.BlockSpec((D_H, 16 * D_H), lambda j: (0, 0)),
        ],
        out_specs=pl.BlockSpec((bm, 16 * D_H), lambda j: (j, 0)),
        out_shape=jax.ShapeDtypeStruct((nrow, 16 * D_H), jnp.float32),
    )(bond2, w2)


def _y12_body(x_ref, a_ref, b_ref, bm_ref, y1_ref, y2_ref):
    x = x_ref[...]
    y1_ref[...] = jnp.dot(x, a_ref[...], preferred_element_type=jnp.float32)
    y2_ref[...] = (
        jnp.dot(x, b_ref[...], preferred_element_type=jnp.float32) + bm_ref[...]
    )


def _y12(x, a, b, bm_i):
    bmr = 2000
    grid = (N_NODES // bmr,)
    return pl.pallas_call(
        _y12_body,
        grid=grid,
        in_specs=[
            pl.BlockSpec((bmr, D_H), lambda i: (i, 0)),
            pl.BlockSpec((D_H, D_H), lambda i: (0, 0)),
            pl.BlockSpec((D_H, D_H), lambda i: (0, 0)),
            pl.BlockSpec((1, D_H), lambda i: (0, 0)),
        ],
        out_specs=[
            pl.BlockSpec((bmr, D_H), lambda i: (i, 0)),
            pl.BlockSpec((bmr, D_H), lambda i: (i, 0)),
        ],
        out_shape=[
            jax.ShapeDtypeStruct((N_NODES, D_H), jnp.float32),
            jax.ShapeDtypeStruct((N_NODES, D_H), jnp.float32),
        ],
    )(x, a, b, bm_i.reshape(1, D_H))


def _update_body(x_ref, p0_ref, p1_ref, wu_ref, bu_ref, o_ref):
    agg = p0_ref[...] + p1_ref[...]
    acc = jnp.dot(agg, wu_ref[...], preferred_element_type=jnp.float32)
    o_ref[...] = jnp.maximum(x_ref[...] + acc + bu_ref[...], 0.0)


def _update(x, p0, p1, wu, bu):
    bmr = 2000
    grid = (N_NODES // bmr,)
    return pl.pallas_call(
        _update_body,
        grid=grid,
        in_specs=[
            pl.BlockSpec((bmr, D_H), lambda i: (i, 0)),
            pl.BlockSpec((bmr, D_H), lambda i: (i, 0)),
            pl.BlockSpec((bmr, D_H), lambda i: (i, 0)),
            pl.BlockSpec((D_H, D_H), lambda i: (0, 0)),
            pl.BlockSpec((1, D_H), lambda i: (0, 0)),
        ],
        out_specs=pl.BlockSpec((bmr, D_H), lambda i: (i, 0)),
        out_shape=jax.ShapeDtypeStruct((N_NODES, D_H), jnp.float32),
    )(x, p0, p1, wu, bu.reshape(1, D_H))


def _score_body(x_ref, w_ref, b_ref, o_ref):
    acc = jnp.dot(x_ref[...], w_ref[...], preferred_element_type=jnp.float32)
    o_ref[...] = jax.nn.sigmoid(acc + b_ref[...])


def _score(x, w_pad, b_pad):
    return pl.pallas_call(
        _score_body,
        in_specs=[
            pl.BlockSpec((N_NODES, D_H), lambda: (0, 0)),
            pl.BlockSpec((D_H, 8), lambda: (0, 0)),
            pl.BlockSpec((1, 8), lambda: (0, 0)),
        ],
        out_specs=pl.BlockSpec((N_NODES, 8), lambda: (0, 0)),
        out_shape=jax.ShapeDtypeStruct((N_NODES, 8), jnp.float32),
    )(x, w_pad, b_pad)


def kernel(atom_feature, edge_index, bond_feature, node2graph, b_factor,
           W_in, b_in, Wm, bm, Wu, bu, W_out, b_out):
    eidx = jnp.stack(
        [
            edge_index[0].reshape(NW, NCHUNK, CHUNK),
            edge_index[1].reshape(NW, NCHUNK, CHUNK),
        ],
        axis=2,
    )

    x = _proj(atom_feature, W_in, b_in)

    bond2 = bond_feature.reshape(N_EDGES // 16, 16 * D_EDGE)
    eye16 = jnp.eye(16, dtype=jnp.float32)
    e_all = [
        _edge_proj(bond2, jnp.kron(eye16, Wm[i, 2 * D_H:, :])).reshape(
            N_EDGES, D_H
        )
        for i in range(N_BLOCK)
    ]

    for i in range(N_BLOCK):
        y1, y2 = _y12(x, Wm[i, :D_H], Wm[i, D_H:2 * D_H], bm[i])
        parts = _sc_agg(y1, y2, e_all[i], eidx)
        x = _update(x, parts[:N_NODES], parts[N_PAD:N_PAD + N_NODES], Wu[i], bu[i])

    w_pad = jnp.pad(W_out, ((0, 0), (0, 7)))
    b_pad = jnp.pad(b_out.reshape(1, 1), ((0, 0), (0, 7)))
    out8 = _score(x, w_pad, b_pad)
    output = out8[:, 0]
    labels = b_factor
    return (output, labels)
